# Initial kernel scaffold; baseline (speedup 1.0000x reference)
import jax
import jax.numpy as jnp
from jax.experimental import pallas as pl


def kernel(source_models, origins):
    raise NotImplementedError("write your pallas kernel here")


# speedup vs baseline: 2.0194x; 2.0194x over previous
import functools

import jax
import jax.numpy as jnp
from jax import lax
from jax.experimental import pallas as pl
from jax.experimental.pallas import tpu as pltpu
from jax.experimental.pallas import tpu_sc as plsc

N_SRC = 64
C = 3
P = 128
H = 2048
W = 2048
SY = 16
NC = 2
NS = 16
NW = NC * NS
N_SLABS = H // SY
ROUNDS = N_SLABS // NW
SLAB_WORDS = C * SY * W
STAGE_WORDS = C * SY * P


def _sc_scatter(patch_flat, orig_flat):
    mesh = plsc.VectorSubcoreMesh(core_axis_name="c", subcore_axis_name="s")

    @functools.partial(
        pl.kernel,
        out_type=jax.ShapeDtypeStruct((C * H * W,), jnp.float32),
        mesh=mesh,
        scratch_types=[
            pltpu.VMEM((SLAB_WORDS,), jnp.float32),
            pltpu.VMEM((STAGE_WORDS,), jnp.float32),
            pltpu.VMEM((2 * N_SRC + 32,), jnp.int32),
            pltpu.SemaphoreType.DMA,
        ],
    )
    def body(patch_hbm, orig_hbm, out_hbm, slab, stage, orig_v, sem):
        wid = lax.axis_index("s") * NC + lax.axis_index("c")
        pltpu.sync_copy(orig_hbm, orig_v)
        zeros16 = jnp.zeros((16,), jnp.float32)

        for r in range(ROUNDS):
            slab_id = r * NW + wid
            y0 = slab_id * SY

            def zero_body(j, _):
                slab[pl.ds(j * 16, 16)] = zeros16
                return 0

            lax.fori_loop(0, SLAB_WORDS // 16, zero_body, 0)

            def src_body(i, _):
                yx = orig_v[pl.ds(2 * i, 16)]
                y = yx[0]
                x = yx[1]
                dy = y0 - y
                overlaps = jnp.logical_and(dy >= -(SY - 1), dy <= P - 1)

                @pl.when(overlaps)
                def _():
                    fs = jnp.clip(dy, 0, P - SY)
                    copies = []
                    for c in range(C):
                        src = patch_hbm.at[
                            pl.ds(((i * C + c) * P + fs) * P, SY * P)
                        ]
                        dst = stage.at[pl.ds(c * SY * P, SY * P)]
                        copies.append(pltpu.async_copy(src, dst, sem))
                    for cp in copies:
                        cp.wait()

                    def row_body(rr, _):
                        q = rr + dy

                        @pl.when(jnp.logical_and(q >= 0, q <= P - 1))
                        def _():
                            srow = q - fs
                            for c in range(C):
                                sbase = (c * SY + srow) * P
                                dbase = (c * SY + rr) * W + x
                                for j in range(P // 16):
                                    v = stage[pl.ds(sbase + j * 16, 16)]
                                    plsc.addupdate(
                                        slab.at[pl.ds(dbase + j * 16, 16)], v
                                    )

                        return 0

                    lax.fori_loop(0, SY, row_body, 0)

                return 0

            lax.fori_loop(0, N_SRC, src_body, 0)

            out_copies = []
            for c in range(C):
                src = slab.at[pl.ds(c * SY * W, SY * W)]
                dst = out_hbm.at[pl.ds((c * H + y0) * W, SY * W)]
                out_copies.append(pltpu.async_copy(src, dst, sem))
            for cp in out_copies:
                cp.wait()

    return body(patch_flat, orig_flat)


def kernel(source_models, origins):
    patch_flat = source_models.reshape(-1)
    orig_flat = jnp.pad(origins.reshape(-1).astype(jnp.int32), (0, 32))
    out = _sc_scatter(patch_flat, orig_flat)
    return out.reshape(C, H, W)


# speedup vs baseline: 3.1160x; 1.5431x over previous
import functools

import jax
import jax.numpy as jnp
from jax import lax
from jax.experimental import pallas as pl
from jax.experimental.pallas import tpu as pltpu
from jax.experimental.pallas import tpu_sc as plsc

N_SRC = 64
C = 3
P = 128
H = 2048
W = 2048
SY = 16
NC = 2
NS = 16
NW = NC * NS
N_SLABS = H // SY
ROUNDS = N_SLABS // NW
SLAB_WORDS = C * SY * W
STAGE_WORDS = C * SY * P


def _sc_scatter(patch_flat, orig_flat):
    mesh = plsc.VectorSubcoreMesh(core_axis_name="c", subcore_axis_name="s")

    @functools.partial(
        pl.kernel,
        out_type=jax.ShapeDtypeStruct((C * H * W,), jnp.float32),
        mesh=mesh,
        scratch_types=[
            pltpu.VMEM((SLAB_WORDS,), jnp.float32),
            pltpu.VMEM((STAGE_WORDS,), jnp.float32),
            pltpu.VMEM((2 * N_SRC + 32,), jnp.int32),
            pltpu.SemaphoreType.DMA,
        ],
    )
    def body(patch_hbm, orig_hbm, out_hbm, slab, stage, orig_v, sem):
        wid = lax.axis_index("s") * NC + lax.axis_index("c")
        pltpu.sync_copy(orig_hbm, orig_v)
        zeros16 = jnp.zeros((16,), jnp.float32)

        for r in range(ROUNDS):
            slab_id = r * NW + wid
            y0 = slab_id * SY

            def zero_body(j, _):
                for u in range(16):
                    slab[pl.ds(j * 256 + u * 16, 16)] = zeros16
                return 0

            lax.fori_loop(0, SLAB_WORDS // 256, zero_body, 0)

            def src_body(i, _):
                yx = orig_v[pl.ds(2 * i, 16)]
                y = yx[0]
                x = yx[1]
                dy = y0 - y
                overlaps = jnp.logical_and(dy >= -(SY - 1), dy <= P - 1)

                @pl.when(overlaps)
                def _():
                    fs = jnp.clip(dy, 0, P - SY)
                    copies = []
                    for c in range(C):
                        src = patch_hbm.at[
                            pl.ds(((i * C + c) * P + fs) * P, SY * P)
                        ]
                        dst = stage.at[pl.ds(c * SY * P, SY * P)]
                        copies.append(pltpu.async_copy(src, dst, sem))
                    for cp in copies:
                        cp.wait()

                    def row_body(rr, _):
                        q = rr + dy

                        @pl.when(jnp.logical_and(q >= 0, q <= P - 1))
                        def _():
                            srow = q - fs
                            for c in range(C):
                                sbase = (c * SY + srow) * P
                                dbase = (c * SY + rr) * W + x
                                for j in range(P // 16):
                                    v = stage[pl.ds(sbase + j * 16, 16)]
                                    plsc.addupdate(
                                        slab.at[pl.ds(dbase + j * 16, 16)], v
                                    )

                        return 0

                    lax.fori_loop(0, SY, row_body, 0)

                return 0

            lax.fori_loop(0, N_SRC, src_body, 0)

            out_copies = []
            for c in range(C):
                src = slab.at[pl.ds(c * SY * W, SY * W)]
                dst = out_hbm.at[pl.ds((c * H + y0) * W, SY * W)]
                out_copies.append(pltpu.async_copy(src, dst, sem))
            for cp in out_copies:
                cp.wait()

    return body(patch_flat, orig_flat)


def kernel(source_models, origins):
    patch_flat = source_models.reshape(-1)
    orig_flat = jnp.pad(origins.reshape(-1).astype(jnp.int32), (0, 32))
    out = _sc_scatter(patch_flat, orig_flat)
    return out.reshape(C, H, W)


# speedup vs baseline: 4.4445x; 1.4264x over previous
import functools

import jax
import jax.numpy as jnp
from jax import lax
from jax.experimental import pallas as pl
from jax.experimental.pallas import tpu as pltpu
from jax.experimental.pallas import tpu_sc as plsc

N_SRC = 64
C = 3
P = 128
H = 2048
W = 2048
SY = 16
NC = 2
NS = 16
NW = NC * NS
N_SLABS = H // SY
ROUNDS = N_SLABS // NW
SLAB_WORDS = C * SY * W
STAGE_WORDS = C * SY * P


def _sc_scatter(patch_flat, orig_flat):
    mesh = plsc.VectorSubcoreMesh(core_axis_name="c", subcore_axis_name="s")

    @functools.partial(
        pl.kernel,
        out_type=jax.ShapeDtypeStruct((C, H, W), jnp.float32),
        mesh=mesh,
        scratch_types=[
            pltpu.VMEM((SLAB_WORDS,), jnp.float32),
            pltpu.VMEM((STAGE_WORDS,), jnp.float32),
            pltpu.VMEM((2 * N_SRC + 32,), jnp.int32),
            pltpu.SemaphoreType.DMA,
        ],
    )
    def body(patch_hbm, orig_hbm, out_hbm, slab, stage, orig_v, sem):
        wid = lax.axis_index("s") * NC + lax.axis_index("c")
        pltpu.sync_copy(orig_hbm, orig_v)
        zeros16 = jnp.zeros((16,), jnp.float32)

        for r in range(ROUNDS):
            slab_id = r * NW + wid
            y0 = slab_id * SY

            def zero_body(j, _):
                for u in range(16):
                    slab[pl.ds(j * 256 + u * 16, 16)] = zeros16
                return 0

            lax.fori_loop(0, SLAB_WORDS // 256, zero_body, 0)

            def src_body(i, _):
                yx = orig_v[pl.ds(2 * i, 16)]
                y = yx[0]
                x = yx[1]
                dy = y0 - y
                overlaps = jnp.logical_and(dy >= -(SY - 1), dy <= P - 1)

                @pl.when(overlaps)
                def _():
                    fs = jnp.clip(dy, 0, P - SY)
                    copies = []
                    for c in range(C):
                        src = patch_hbm.at[
                            pl.ds(((i * C + c) * P + fs) * P, SY * P)
                        ]
                        dst = stage.at[pl.ds(c * SY * P, SY * P)]
                        copies.append(pltpu.async_copy(src, dst, sem))
                    for cp in copies:
                        cp.wait()

                    def row_body(rr, _):
                        q = rr + dy

                        @pl.when(jnp.logical_and(q >= 0, q <= P - 1))
                        def _():
                            srow = q - fs
                            for c in range(C):
                                sbase = (c * SY + srow) * P
                                dbase = (c * SY + rr) * W + x
                                for j in range(P // 16):
                                    v = stage[pl.ds(sbase + j * 16, 16)]
                                    plsc.addupdate(
                                        slab.at[pl.ds(dbase + j * 16, 16)], v
                                    )

                        return 0

                    lax.fori_loop(0, SY, row_body, 0)

                return 0

            lax.fori_loop(0, N_SRC, src_body, 0)

            def wb_body(j, _):
                c = j // SY
                rr = j % SY
                src = slab.at[pl.ds((c * SY + rr) * W, W)]
                dst = out_hbm.at[c, y0 + rr, :]
                pltpu.async_copy(src, dst, sem)
                return 0

            lax.fori_loop(0, C * SY, wb_body, 0)

            def wb_wait(j, _):
                pltpu.make_async_copy(
                    slab.at[pl.ds(0, W)], out_hbm.at[0, 0, :], sem
                ).wait()
                return 0

            lax.fori_loop(0, C * SY, wb_wait, 0)

    return body(patch_flat, orig_flat)


def kernel(source_models, origins):
    patch_flat = source_models.reshape(-1)
    orig_flat = jnp.pad(origins.reshape(-1).astype(jnp.int32), (0, 32))
    return _sc_scatter(patch_flat, orig_flat)
